# baseline (speedup 1.0000x reference)
import functools

import jax
import jax.numpy as jnp
from jax import lax
from jax.experimental import pallas as pl
from jax.experimental.pallas import tpu as pltpu
from jax.experimental.pallas import tpu_sc as plsc

NUM_TOKENS = 65536
D = 1024
B = 128
HW = 1024
NC = 2
NS = 16
NW = NC * NS
PW = HW // NW
LANES = 16

_mesh = plsc.VectorSubcoreMesh(core_axis_name="c", subcore_axis_name="s")


@functools.partial(
    pl.kernel,
    mesh=_mesh,
    out_type=jax.ShapeDtypeStruct((B, HW, D), jnp.float32),
    scratch_types=[
        pltpu.VMEM((B, PW), jnp.int32),
        pltpu.VMEM((PW, D), jnp.float32),
        pltpu.VMEM((PW, D), jnp.float32),
        pltpu.SemaphoreType.DMA,
    ],
)
def _embed(idx_hbm, table_hbm, pos_hbm, out_hbm, idxv, posv, gv, sem):
    wid = lax.axis_index("s") * NC + lax.axis_index("c")
    pltpu.sync_copy(idx_hbm.at[wid], idxv)
    pltpu.sync_copy(pos_hbm.at[pl.ds(wid * PW, PW), :], posv)

    def body(b, carry):
        pltpu.async_copy(table_hbm.at[idxv.at[b]], gv, sem).wait()

        def add_row(r, c2):
            def add_col(c, c3):
                sl = pl.ds(c * LANES, LANES)
                gv[r, sl] = gv[r, sl] + posv[r, sl]
                return c3
            return lax.fori_loop(0, D // LANES, add_col, c2)
        lax.fori_loop(0, PW, add_row, carry)

        pltpu.sync_copy(gv, out_hbm.at[b, pl.ds(wid * PW, PW), :])
        return carry
    lax.fori_loop(0, B, body, 0)


def kernel(token_indices, token_embedding, pos_embedding):
    b, h, w = token_indices.shape
    idx_t = (
        token_indices.astype(jnp.int32)
        .reshape(B, NW, PW)
        .transpose(1, 0, 2)
    )
    pos2d = pos_embedding.reshape(HW, D)
    return _embed(idx_t, token_embedding, pos2d)


# speedup vs baseline: 2.9417x; 2.9417x over previous
import functools

import jax
import jax.numpy as jnp
from jax import lax
from jax.experimental import pallas as pl
from jax.experimental.pallas import tpu as pltpu
from jax.experimental.pallas import tpu_sc as plsc

NUM_TOKENS = 65536
D = 1024
B = 128
HW = 1024
NC = 2
NS = 16
NW = NC * NS
PW = HW // NW
LANES = 16
VPR = D // LANES

_mesh = plsc.VectorSubcoreMesh(core_axis_name="c", subcore_axis_name="s")


@functools.partial(
    pl.kernel,
    mesh=_mesh,
    out_type=jax.ShapeDtypeStruct((B, HW, D), jnp.float32),
    scratch_types=[
        pltpu.VMEM((B, PW), jnp.int32),
        pltpu.VMEM((PW, D), jnp.float32),
        pltpu.VMEM((PW, D), jnp.float32),
        pltpu.VMEM((PW, D), jnp.float32),
        pltpu.SemaphoreType.DMA,
        pltpu.SemaphoreType.DMA,
        pltpu.SemaphoreType.DMA,
        pltpu.SemaphoreType.DMA,
    ],
)
def _embed(idx_hbm, table_hbm, pos_hbm, out_hbm, idxv, posv, g0, g1,
           sg0, sg1, ss0, ss1):
    wid = lax.axis_index("s") * NC + lax.axis_index("c")
    pltpu.sync_copy(idx_hbm.at[wid], idxv)
    pltpu.sync_copy(pos_hbm.at[pl.ds(wid * PW, PW), :], posv)

    def start_gather(b, gbuf, sem):
        pltpu.make_async_copy(table_hbm.at[idxv.at[b]], gbuf, sem).start()

    def wait_gather(gbuf, sem):
        pltpu.make_async_copy(table_hbm.at[idxv.at[0]], gbuf, sem).wait()

    def start_scatter(b, gbuf, sem):
        pltpu.make_async_copy(
            gbuf, out_hbm.at[b, pl.ds(wid * PW, PW), :], sem).start()

    def wait_scatter(gbuf, sem):
        pltpu.make_async_copy(
            gbuf, out_hbm.at[0, pl.ds(wid * PW, PW), :], sem).wait()

    def add_pos(gbuf):
        def add_row(r, carry):
            for c in range(VPR):
                sl = pl.ds(c * LANES, LANES)
                plsc.addupdate(gbuf.at[r, sl], posv[r, sl])
            return carry
        lax.fori_loop(0, PW, add_row, 0)

    start_gather(0, g0, sg0)
    start_gather(1, g1, sg1)

    def body(i, carry):
        b0 = 2 * i
        b1 = 2 * i + 1
        wait_gather(g0, sg0)
        add_pos(g0)
        start_scatter(b0, g0, ss0)
        wait_gather(g1, sg1)
        add_pos(g1)
        wait_scatter(g0, ss0)
        start_gather(b0 + 2, g0, sg0)
        start_scatter(b1, g1, ss1)
        wait_scatter(g1, ss1)
        start_gather(b1 + 2, g1, sg1)
        return carry
    lax.fori_loop(0, B // 2 - 1, body, 0)

    wait_gather(g0, sg0)
    add_pos(g0)
    start_scatter(B - 2, g0, ss0)
    wait_gather(g1, sg1)
    add_pos(g1)
    start_scatter(B - 1, g1, ss1)
    wait_scatter(g0, ss0)
    wait_scatter(g1, ss1)


def kernel(token_indices, token_embedding, pos_embedding):
    b, h, w = token_indices.shape
    idx_t = (
        token_indices.astype(jnp.int32)
        .reshape(B, NW, PW)
        .transpose(1, 0, 2)
    )
    pos2d = pos_embedding.reshape(HW, D)
    return _embed(idx_t, token_embedding, pos2d)
